# baseline (speedup 1.0000x reference)
import functools

import jax
import jax.numpy as jnp
from jax import lax
from jax.experimental import pallas as pl
from jax.experimental.pallas import tpu as pltpu
from jax.experimental.pallas import tpu_sc as plsc

F32 = jnp.float32
I32 = jnp.int32

N0 = 10000
N0P = 10240
E1 = 160000
E2 = 65536
N1 = 4096
N2 = 1024


def _proj_body(x_ref, wl_ref, wr_ref, out_l_ref, out_r_ref):
    x = x_ref[...]
    out_l_ref[...] = jnp.dot(x, wl_ref[...], preferred_element_type=F32)
    out_r_ref[...] = jnp.dot(x, wr_ref[...], preferred_element_type=F32)


def _proj(x, wl, wr):
    return pl.pallas_call(
        _proj_body,
        grid=(10,),
        in_specs=[
            pl.BlockSpec((1000, 256), lambda i: (i, 0)),
            pl.BlockSpec((256, 16), lambda i: (0, 0)),
            pl.BlockSpec((256, 16), lambda i: (0, 0)),
        ],
        out_specs=[
            pl.BlockSpec((1000, 16), lambda i: (i, 0)),
            pl.BlockSpec((1000, 16), lambda i: (i, 0)),
        ],
        out_shape=[
            jax.ShapeDtypeStruct((N0, 16), F32),
            jax.ShapeDtypeStruct((N0, 16), F32),
        ],
    )(x, wl, wr)


_MESH = plsc.VectorSubcoreMesh(core_axis_name="c", subcore_axis_name="s")
_SC_PARAMS = pltpu.CompilerParams(use_tc_tiling_on_sc=False,
                                  needs_layout_passes=False)


@functools.partial(
    pl.kernel,
    mesh=_MESH,
    out_type=[
        jax.ShapeDtypeStruct((N1, 16), F32),
        jax.ShapeDtypeStruct((2, N0P, 16), F32),
        jax.ShapeDtypeStruct((2, N1, 16), F32),
        jax.ShapeDtypeStruct((2, N1), F32),
    ],
    scratch_types=[
        pltpu.VMEM((256, 16), F32),
        pltpu.VMEM((256,), F32),
        pltpu.VMEM((1024,), F32),
        pltpu.VMEM((1024,), I32),
        pltpu.VMEM((1024,), I32),
        pltpu.VMEM((1000, 16), F32),
        pltpu.VMEM((1000, 16), F32),
        pltpu.VMEM((1024, 16), F32),
        pltpu.VMEM((5000,), I32),
        pltpu.VMEM((5000,), I32),
        pltpu.VMEM_SHARED((N1, 16), F32),
        pltpu.VMEM_SHARED((N1,), F32),
        pltpu.SemaphoreType.DMA,
        pltpu.SemaphoreType.DMA,
        pltpu.SemaphoreType.DMA,
        pltpu.SemaphoreType.DMA,
        pltpu.SemaphoreType.DMA,
    ],
    compiler_params=_SC_PARAMS,
)
def _sc1(pl_hbm, pr_hbm, nid_hbm, src_hbm, dst_hbm,
         root_out, g_out, acc_out, cnt_out,
         zbuf, czbuf, ones_v, nid_v, ridx_v, growA, growB, rrow_v,
         src_v, dst_v, acc_sh, cnt_sh,
         sem_g0, sem_g1, sem_a0, sem_a1, sem_c):
    c = lax.axis_index("c")
    s = lax.axis_index("s")
    w = s * 2 + c

    def fillz(i, _):
        zbuf[i] = jnp.zeros((16,), F32)
        return 0
    lax.fori_loop(0, 256, fillz, 0)

    def fillcz(i, _):
        czbuf[pl.ds(i * 16, 16)] = jnp.zeros((16,), F32)
        return 0
    lax.fori_loop(0, 16, fillcz, 0)

    def fill1(i, _):
        ones_v[pl.ds(i * 16, 16)] = jnp.full((16,), 1.0, F32)
        return 0
    lax.fori_loop(0, 64, fill1, 0)

    zd0 = pltpu.async_copy(zbuf, acc_sh.at[pl.ds(s * 256, 256)], sem_a0)
    zd1 = pltpu.async_copy(czbuf, cnt_sh.at[pl.ds(s * 256, 256)], sem_a1)

    @pl.when(s < 9)
    def _():
        pltpu.sync_copy(nid_hbm.at[pl.ds(s * 1024, 1024)], nid_v)
        pltpu.sync_copy(pl_hbm.at[nid_v], rrow_v)
        pltpu.sync_copy(rrow_v, g_out.at[c, pl.ds(s * 1024, 1024)])

    @pl.when(s == 9)
    def _():
        pltpu.sync_copy(nid_hbm.at[pl.ds(9216, 784)], nid_v.at[pl.ds(0, 784)])
        pltpu.sync_copy(pl_hbm.at[nid_v.at[pl.ds(0, 784)]],
                        rrow_v.at[pl.ds(0, 784)])
        pltpu.sync_copy(rrow_v.at[pl.ds(0, 784)],
                        g_out.at[c, pl.ds(9216, 784)])

    @pl.when(w < 4)
    def _():
        pltpu.sync_copy(nid_hbm.at[pl.ds(w * 1024, 1024)], ridx_v)
        pltpu.sync_copy(pr_hbm.at[ridx_v], rrow_v)
        pltpu.sync_copy(rrow_v, root_out.at[pl.ds(w * 1024, 1024)])

    zd0.wait()
    zd1.wait()
    plsc.subcore_barrier()

    pltpu.sync_copy(src_hbm.at[pl.ds(w * 5000, 5000)], src_v)
    pltpu.sync_copy(dst_hbm.at[pl.ds(w * 5000, 5000)], dst_v)

    gtab = g_out.at[c]
    bufs = (growA, growB)
    gsems = (sem_g0, sem_g1)
    asems = (sem_a0, sem_a1)
    gd = [None] * 5
    ad = [None] * 5
    cd = [None] * 5
    gd[0] = pltpu.async_copy(gtab.at[src_v.at[pl.ds(0, 1000)]], bufs[0],
                             gsems[0])
    for j in range(5):
        if j + 1 < 5:
            if j - 1 >= 0:
                ad[j - 1].wait()
            gd[j + 1] = pltpu.async_copy(
                gtab.at[src_v.at[pl.ds((j + 1) * 1000, 1000)]],
                bufs[(j + 1) % 2], gsems[(j + 1) % 2])
        gd[j].wait()
        ad[j] = pltpu.async_copy(
            bufs[j % 2], acc_sh.at[dst_v.at[pl.ds(j * 1000, 1000)]],
            asems[j % 2], add=True)
        cd[j] = pltpu.async_copy(
            ones_v.at[pl.ds(0, 1000)],
            cnt_sh.at[dst_v.at[pl.ds(j * 1000, 1000)]], sem_c, add=True)
    ad[3].wait()
    ad[4].wait()
    for j in range(5):
        cd[j].wait()

    plsc.subcore_barrier()

    pltpu.sync_copy(acc_sh.at[pl.ds(s * 256, 256)], growA.at[pl.ds(0, 256)])
    pltpu.sync_copy(growA.at[pl.ds(0, 256)], acc_out.at[c, pl.ds(s * 256, 256)])
    pltpu.sync_copy(cnt_sh.at[pl.ds(s * 256, 256)], czbuf)
    pltpu.sync_copy(czbuf, cnt_out.at[c, pl.ds(s * 256, 256)])


@functools.partial(
    pl.kernel,
    mesh=_MESH,
    out_type=[
        jax.ShapeDtypeStruct((N2, 16), F32),
        jax.ShapeDtypeStruct((2, N2, 16), F32),
        jax.ShapeDtypeStruct((2, N2, 16), F32),
    ],
    scratch_types=[
        pltpu.VMEM((64, 16), F32),
        pltpu.VMEM((256, 16), F32),
        pltpu.VMEM((256, 16), F32),
        pltpu.VMEM((256,), F32),
        pltpu.VMEM((256,), F32),
        pltpu.VMEM((256, 16), F32),
        pltpu.VMEM((256, 16), F32),
        pltpu.VMEM((16,), F32),
        pltpu.VMEM((1024, 16), F32),
        pltpu.VMEM((1024, 16), F32),
        pltpu.VMEM((2048,), I32),
        pltpu.VMEM((2048,), I32),
        pltpu.VMEM((1024, 16), F32),
        pltpu.VMEM_SHARED((N1, 16), F32),
        pltpu.VMEM_SHARED((N2, 16), F32),
        pltpu.VMEM_SHARED((N2, 16), F32),
        pltpu.SemaphoreType.DMA,
        pltpu.SemaphoreType.DMA,
        pltpu.SemaphoreType.DMA,
        pltpu.SemaphoreType.DMA,
        pltpu.SemaphoreType.DMA,
    ],
    compiler_params=_SC_PARAMS,
)
def _sc2(acc1_hbm, cnt1_hbm, root_hbm, b_hbm, src_hbm, dst_hbm,
         h1q_out, acc_out, cnt_out,
         zbuf, a0_v, a1_v, c0_v, c1_v, root_v, hbuf, b_v,
         growA, growB, src_v, dst_v, ones_v, h_sh, acc_sh, cnt_sh,
         sem_g0, sem_g1, sem_a0, sem_a1, sem_c):
    c = lax.axis_index("c")
    s = lax.axis_index("s")
    w = s * 2 + c

    def fill(i, _):
        zbuf[i] = jnp.zeros((16,), F32)
        return 0
    lax.fori_loop(0, 64, fill, 0)

    def fill1(i, _):
        ones_v[i] = jnp.full((16,), 1.0, F32)
        return 0
    lax.fori_loop(0, 1024, fill1, 0)

    zd0 = pltpu.async_copy(zbuf, acc_sh.at[pl.ds(s * 64, 64)], sem_a0)
    zd1 = pltpu.async_copy(zbuf, cnt_sh.at[pl.ds(s * 64, 64)], sem_a1)

    r0 = s * 256
    d0 = pltpu.async_copy(acc1_hbm.at[0, pl.ds(r0, 256)], a0_v, sem_c)
    d1 = pltpu.async_copy(acc1_hbm.at[1, pl.ds(r0, 256)], a1_v, sem_c)
    d2 = pltpu.async_copy(cnt1_hbm.at[0, pl.ds(r0, 256)], c0_v, sem_c)
    d3 = pltpu.async_copy(cnt1_hbm.at[1, pl.ds(r0, 256)], c1_v, sem_c)
    d4 = pltpu.async_copy(root_hbm.at[pl.ds(r0, 256)], root_v, sem_c)
    d5 = pltpu.async_copy(b_hbm, b_v, sem_c)
    for d in (d0, d1, d2, d3, d4, d5):
        d.wait()

    def cadd(i, _):
        o = pl.ds(i * 16, 16)
        c0_v[o] = c0_v[o] + c1_v[o]
        return 0
    lax.fori_loop(0, 16, cadd, 0)

    def hrow(i, _):
        n = jnp.maximum(plsc.load_gather(c0_v, [jnp.full((16,), i, I32)]), 1.0)
        h = jnp.maximum((a0_v[i] + a1_v[i]) / n + root_v[i] + b_v[...], 0.0)
        hbuf[i] = h
        return 0
    lax.fori_loop(0, 256, hrow, 0)

    pltpu.sync_copy(hbuf, h_sh.at[pl.ds(r0, 256)])

    @pl.when(s < 4)
    def _():
        pltpu.sync_copy(hbuf, h1q_out.at[pl.ds(r0, 256)])

    zd0.wait()
    zd1.wait()
    plsc.subcore_barrier()

    pltpu.sync_copy(src_hbm.at[pl.ds(w * 2048, 2048)], src_v)
    pltpu.sync_copy(dst_hbm.at[pl.ds(w * 2048, 2048)], dst_v)

    g0 = pltpu.async_copy(h_sh.at[src_v.at[pl.ds(0, 1024)]], growA, sem_g0)
    g1 = pltpu.async_copy(h_sh.at[src_v.at[pl.ds(1024, 1024)]], growB, sem_g1)
    g0.wait()
    a0 = pltpu.async_copy(growA, acc_sh.at[dst_v.at[pl.ds(0, 1024)]],
                          sem_a0, add=True)
    k0 = pltpu.async_copy(ones_v, cnt_sh.at[dst_v.at[pl.ds(0, 1024)]],
                          sem_c, add=True)
    g1.wait()
    a1 = pltpu.async_copy(growB, acc_sh.at[dst_v.at[pl.ds(1024, 1024)]],
                          sem_a1, add=True)
    k1 = pltpu.async_copy(ones_v, cnt_sh.at[dst_v.at[pl.ds(1024, 1024)]],
                          sem_c, add=True)
    for d in (a0, k0, a1, k1):
        d.wait()

    plsc.subcore_barrier()

    r = s * 64
    pltpu.sync_copy(acc_sh.at[pl.ds(r, 64)], growA.at[pl.ds(0, 64)])
    pltpu.sync_copy(growA.at[pl.ds(0, 64)], acc_out.at[c, pl.ds(r, 64)])
    pltpu.sync_copy(cnt_sh.at[pl.ds(r, 64)], growB.at[pl.ds(0, 64)])
    pltpu.sync_copy(growB.at[pl.ds(0, 64)], cnt_out.at[c, pl.ds(r, 64)])


def _comb2_body(acc_ref, cnt_ref, root_ref, wl_ref, wr_ref, b_ref, out_ref):
    sums = acc_ref[0] + acc_ref[1]
    cnt = jnp.maximum(cnt_ref[0] + cnt_ref[1], 1.0)
    mean = sums / cnt
    z = (jnp.dot(mean, wl_ref[...], preferred_element_type=F32)
         + jnp.dot(root_ref[...], wr_ref[...], preferred_element_type=F32)
         + b_ref[...][None, :])
    m = jnp.max(z, axis=1, keepdims=True)
    lse = m + jnp.log(jnp.sum(jnp.exp(z - m), axis=1, keepdims=True))
    out_ref[...] = z - lse


def _comb2(acc, cnt, root, wl2, wr2, b2):
    return pl.pallas_call(
        _comb2_body,
        out_shape=jax.ShapeDtypeStruct((N2, 41), F32),
    )(acc, cnt, root, wl2, wr2, b2)


def kernel(x, n_id, src1, dst1, src2, dst2, W_l1, W_r1, b1, W_l2, W_r2, b2):
    p_l, p_r = _proj(x, W_l1, W_r1)

    root1, _g, acc1, cnt1 = _sc1(p_l, p_r, n_id.astype(I32),
                                 src1.astype(I32), dst1.astype(I32))

    h1q, acc2, cnt2 = _sc2(acc1, cnt1, root1, b1,
                           src2.astype(I32), dst2.astype(I32))

    return _comb2(acc2, cnt2, h1q, W_l2, W_r2, b2)


# speedup vs baseline: 1.1203x; 1.1203x over previous
import functools

import jax
import jax.numpy as jnp
from jax import lax
from jax.experimental import pallas as pl
from jax.experimental.pallas import tpu as pltpu
from jax.experimental.pallas import tpu_sc as plsc

F32 = jnp.float32
I32 = jnp.int32

N0 = 10000
N0P = 10240
E1 = 160000
E2 = 65536
N1 = 4096
N2 = 1024


def _proj_body(x_ref, wl_ref, wr_ref, out_l_ref, out_r_ref):
    x = x_ref[...]
    out_l_ref[...] = jnp.dot(x, wl_ref[...], preferred_element_type=F32)
    out_r_ref[...] = jnp.dot(x, wr_ref[...], preferred_element_type=F32)


def _proj(x, wl, wr):
    return pl.pallas_call(
        _proj_body,
        grid=(5,),
        in_specs=[
            pl.BlockSpec((2000, 256), lambda i: (i, 0)),
            pl.BlockSpec((256, 16), lambda i: (0, 0)),
            pl.BlockSpec((256, 16), lambda i: (0, 0)),
        ],
        out_specs=[
            pl.BlockSpec((2000, 16), lambda i: (i, 0)),
            pl.BlockSpec((2000, 16), lambda i: (i, 0)),
        ],
        out_shape=[
            jax.ShapeDtypeStruct((N0, 16), F32),
            jax.ShapeDtypeStruct((N0, 16), F32),
        ],
    )(x, wl, wr)


_MESH = plsc.VectorSubcoreMesh(core_axis_name="c", subcore_axis_name="s")
_SC_PARAMS = pltpu.CompilerParams(use_tc_tiling_on_sc=False,
                                  needs_layout_passes=False)


@functools.partial(
    pl.kernel,
    mesh=_MESH,
    out_type=[
        jax.ShapeDtypeStruct((N1, 16), F32),
        jax.ShapeDtypeStruct((2, N0P, 16), F32),
        jax.ShapeDtypeStruct((2, N1, 16), F32),
        jax.ShapeDtypeStruct((2, N1), F32),
    ],
    scratch_types=[
        pltpu.VMEM((256, 16), F32),
        pltpu.VMEM((256,), F32),
        pltpu.VMEM((1024,), F32),
        pltpu.VMEM((1024,), I32),
        pltpu.VMEM((1024,), I32),
        pltpu.VMEM((1000, 16), F32),
        pltpu.VMEM((1000, 16), F32),
        pltpu.VMEM((1024, 16), F32),
        pltpu.VMEM((5000,), I32),
        pltpu.VMEM((5000,), I32),
        pltpu.VMEM_SHARED((N1, 16), F32),
        pltpu.VMEM_SHARED((N1,), F32),
        pltpu.SemaphoreType.DMA,
        pltpu.SemaphoreType.DMA,
        pltpu.SemaphoreType.DMA,
        pltpu.SemaphoreType.DMA,
        pltpu.SemaphoreType.DMA,
    ],
    compiler_params=_SC_PARAMS,
)
def _sc1(pl_hbm, pr_hbm, nid_hbm, src_hbm, dst_hbm,
         root_out, g_out, acc_out, cnt_out,
         zbuf, czbuf, ones_v, nid_v, ridx_v, growA, growB, rrow_v,
         src_v, dst_v, acc_sh, cnt_sh,
         sem_g0, sem_g1, sem_a0, sem_a1, sem_c):
    c = lax.axis_index("c")
    s = lax.axis_index("s")
    w = s * 2 + c

    def fillz(i, _):
        zbuf[i] = jnp.zeros((16,), F32)
        return 0
    lax.fori_loop(0, 256, fillz, 0)

    def fillcz(i, _):
        czbuf[pl.ds(i * 16, 16)] = jnp.zeros((16,), F32)
        return 0
    lax.fori_loop(0, 16, fillcz, 0)

    def fill1(i, _):
        ones_v[pl.ds(i * 16, 16)] = jnp.full((16,), 1.0, F32)
        return 0
    lax.fori_loop(0, 64, fill1, 0)

    zd0 = pltpu.async_copy(zbuf, acc_sh.at[pl.ds(s * 256, 256)], sem_a0)
    zd1 = pltpu.async_copy(czbuf, cnt_sh.at[pl.ds(s * 256, 256)], sem_a1)

    pd0 = pltpu.async_copy(src_hbm.at[pl.ds(w * 5000, 5000)], src_v, sem_g0)
    pd1 = pltpu.async_copy(dst_hbm.at[pl.ds(w * 5000, 5000)], dst_v, sem_g1)

    @pl.when(s < 9)
    def _():
        pltpu.sync_copy(nid_hbm.at[pl.ds(s * 1024, 1024)], nid_v)
        pltpu.sync_copy(pl_hbm.at[nid_v], rrow_v)
        pltpu.sync_copy(rrow_v, g_out.at[c, pl.ds(s * 1024, 1024)])

    @pl.when(s == 9)
    def _():
        pltpu.sync_copy(nid_hbm.at[pl.ds(9216, 784)], nid_v.at[pl.ds(0, 784)])
        pltpu.sync_copy(pl_hbm.at[nid_v.at[pl.ds(0, 784)]],
                        rrow_v.at[pl.ds(0, 784)])
        pltpu.sync_copy(rrow_v.at[pl.ds(0, 784)],
                        g_out.at[c, pl.ds(9216, 784)])

    @pl.when((s >= 10) & (s < 12))
    def _():
        q = (s - 10) * 2 + c
        pltpu.sync_copy(nid_hbm.at[pl.ds(q * 1024, 1024)], ridx_v)
        pltpu.sync_copy(pr_hbm.at[ridx_v], rrow_v)
        pltpu.sync_copy(rrow_v, root_out.at[pl.ds(q * 1024, 1024)])

    zd0.wait()
    zd1.wait()
    pd0.wait()
    pd1.wait()
    plsc.subcore_barrier()

    gtab = g_out.at[c]
    bufs = (growA, growB)
    gsems = (sem_g0, sem_g1)
    asems = (sem_a0, sem_a1)
    gd = [None] * 5
    ad = [None] * 5
    cd = [None] * 5
    gd[0] = pltpu.async_copy(gtab.at[src_v.at[pl.ds(0, 1000)]], bufs[0],
                             gsems[0])
    for j in range(5):
        if j + 1 < 5:
            if j - 1 >= 0:
                ad[j - 1].wait()
            gd[j + 1] = pltpu.async_copy(
                gtab.at[src_v.at[pl.ds((j + 1) * 1000, 1000)]],
                bufs[(j + 1) % 2], gsems[(j + 1) % 2])
        gd[j].wait()
        ad[j] = pltpu.async_copy(
            bufs[j % 2], acc_sh.at[dst_v.at[pl.ds(j * 1000, 1000)]],
            asems[j % 2], add=True)
        cd[j] = pltpu.async_copy(
            ones_v.at[pl.ds(0, 1000)],
            cnt_sh.at[dst_v.at[pl.ds(j * 1000, 1000)]], sem_c, add=True)
    ad[3].wait()
    ad[4].wait()
    for j in range(5):
        cd[j].wait()

    plsc.subcore_barrier()

    pltpu.sync_copy(acc_sh.at[pl.ds(s * 256, 256)], growA.at[pl.ds(0, 256)])
    pltpu.sync_copy(growA.at[pl.ds(0, 256)], acc_out.at[c, pl.ds(s * 256, 256)])
    pltpu.sync_copy(cnt_sh.at[pl.ds(s * 256, 256)], czbuf)
    pltpu.sync_copy(czbuf, cnt_out.at[c, pl.ds(s * 256, 256)])


@functools.partial(
    pl.kernel,
    mesh=_MESH,
    out_type=[
        jax.ShapeDtypeStruct((5, N2, 16), F32),
    ],
    scratch_types=[
        pltpu.VMEM((64, 16), F32),
        pltpu.VMEM((256, 16), F32),
        pltpu.VMEM((256, 16), F32),
        pltpu.VMEM((256,), F32),
        pltpu.VMEM((256,), F32),
        pltpu.VMEM((256, 16), F32),
        pltpu.VMEM((256, 16), F32),
        pltpu.VMEM((16,), F32),
        pltpu.VMEM((1024, 16), F32),
        pltpu.VMEM((1024, 16), F32),
        pltpu.VMEM((2048,), I32),
        pltpu.VMEM((2048,), I32),
        pltpu.VMEM((1024, 16), F32),
        pltpu.VMEM_SHARED((N1, 16), F32),
        pltpu.VMEM_SHARED((N2, 16), F32),
        pltpu.VMEM_SHARED((N2, 16), F32),
        pltpu.SemaphoreType.DMA,
        pltpu.SemaphoreType.DMA,
        pltpu.SemaphoreType.DMA,
        pltpu.SemaphoreType.DMA,
        pltpu.SemaphoreType.DMA,
    ],
    compiler_params=_SC_PARAMS,
)
def _sc2(acc1_hbm, cnt1_hbm, root_hbm, b_hbm, src_hbm, dst_hbm,
         l2_out,
         zbuf, a0_v, a1_v, c0_v, c1_v, root_v, hbuf, b_v,
         growA, growB, src_v, dst_v, ones_v, h_sh, acc_sh, cnt_sh,
         sem_g0, sem_g1, sem_a0, sem_a1, sem_c):
    c = lax.axis_index("c")
    s = lax.axis_index("s")
    w = s * 2 + c

    def fill(i, _):
        zbuf[i] = jnp.zeros((16,), F32)
        return 0
    lax.fori_loop(0, 64, fill, 0)

    def fill1(i, _):
        ones_v[i] = jnp.full((16,), 1.0, F32)
        return 0
    lax.fori_loop(0, 1024, fill1, 0)

    zd0 = pltpu.async_copy(zbuf, acc_sh.at[pl.ds(s * 64, 64)], sem_a0)
    zd1 = pltpu.async_copy(zbuf, cnt_sh.at[pl.ds(s * 64, 64)], sem_a1)

    r0 = s * 256
    d0 = pltpu.async_copy(acc1_hbm.at[0, pl.ds(r0, 256)], a0_v, sem_c)
    d1 = pltpu.async_copy(acc1_hbm.at[1, pl.ds(r0, 256)], a1_v, sem_c)
    d2 = pltpu.async_copy(cnt1_hbm.at[0, pl.ds(r0, 256)], c0_v, sem_c)
    d3 = pltpu.async_copy(cnt1_hbm.at[1, pl.ds(r0, 256)], c1_v, sem_c)
    d4 = pltpu.async_copy(root_hbm.at[pl.ds(r0, 256)], root_v, sem_c)
    d5 = pltpu.async_copy(b_hbm, b_v, sem_c)
    for d in (d0, d1, d2, d3, d4, d5):
        d.wait()

    def cadd(i, _):
        o = pl.ds(i * 16, 16)
        c0_v[o] = c0_v[o] + c1_v[o]
        return 0
    lax.fori_loop(0, 16, cadd, 0)

    def hrow(i, _):
        n = jnp.maximum(plsc.load_gather(c0_v, [jnp.full((16,), i, I32)]), 1.0)
        h = jnp.maximum((a0_v[i] + a1_v[i]) / n + root_v[i] + b_v[...], 0.0)
        hbuf[i] = h
        return 0
    lax.fori_loop(0, 256, hrow, 0)

    pltpu.sync_copy(hbuf, h_sh.at[pl.ds(r0, 256)])

    @pl.when((s < 4) & (c == 0))
    def _():
        pltpu.sync_copy(hbuf, l2_out.at[4, pl.ds(r0, 256)])

    zd0.wait()
    zd1.wait()
    plsc.subcore_barrier()

    pltpu.sync_copy(src_hbm.at[pl.ds(w * 2048, 2048)], src_v)
    pltpu.sync_copy(dst_hbm.at[pl.ds(w * 2048, 2048)], dst_v)

    g0 = pltpu.async_copy(h_sh.at[src_v.at[pl.ds(0, 1024)]], growA, sem_g0)
    g1 = pltpu.async_copy(h_sh.at[src_v.at[pl.ds(1024, 1024)]], growB, sem_g1)
    g0.wait()
    a0 = pltpu.async_copy(growA, acc_sh.at[dst_v.at[pl.ds(0, 1024)]],
                          sem_a0, add=True)
    k0 = pltpu.async_copy(ones_v, cnt_sh.at[dst_v.at[pl.ds(0, 1024)]],
                          sem_c, add=True)
    g1.wait()
    a1 = pltpu.async_copy(growB, acc_sh.at[dst_v.at[pl.ds(1024, 1024)]],
                          sem_a1, add=True)
    k1 = pltpu.async_copy(ones_v, cnt_sh.at[dst_v.at[pl.ds(1024, 1024)]],
                          sem_c, add=True)
    for d in (a0, k0, a1, k1):
        d.wait()

    plsc.subcore_barrier()

    r = s * 64
    pltpu.sync_copy(acc_sh.at[pl.ds(r, 64)], growA.at[pl.ds(0, 64)])
    pltpu.sync_copy(growA.at[pl.ds(0, 64)], l2_out.at[c, pl.ds(r, 64)])
    pltpu.sync_copy(cnt_sh.at[pl.ds(r, 64)], growB.at[pl.ds(0, 64)])
    pltpu.sync_copy(growB.at[pl.ds(0, 64)], l2_out.at[2 + c, pl.ds(r, 64)])


def _comb2_body(l2_ref, wl_ref, wr_ref, b_ref, out_ref):
    sums = l2_ref[0] + l2_ref[1]
    cnt = jnp.maximum(l2_ref[2] + l2_ref[3], 1.0)
    mean = sums / cnt
    z = (jnp.dot(mean, wl_ref[...], preferred_element_type=F32)
         + jnp.dot(l2_ref[4], wr_ref[...], preferred_element_type=F32)
         + b_ref[...][None, :])
    m = jnp.max(z, axis=1, keepdims=True)
    lse = m + jnp.log(jnp.sum(jnp.exp(z - m), axis=1, keepdims=True))
    out_ref[...] = z - lse


def _comb2(l2, wl2, wr2, b2):
    return pl.pallas_call(
        _comb2_body,
        out_shape=jax.ShapeDtypeStruct((N2, 41), F32),
    )(l2, wl2, wr2, b2)


def kernel(x, n_id, src1, dst1, src2, dst2, W_l1, W_r1, b1, W_l2, W_r2, b2):
    p_l, p_r = _proj(x, W_l1, W_r1)

    root1, _g, acc1, cnt1 = _sc1(p_l, p_r, n_id.astype(I32),
                                 src1.astype(I32), dst1.astype(I32))

    (l2,) = _sc2(acc1, cnt1, root1, b1,
                 src2.astype(I32), dst2.astype(I32))

    return _comb2(l2, W_l2, W_r2, b2)


# speedup vs baseline: 1.2034x; 1.0742x over previous
import functools

import jax
import jax.numpy as jnp
from jax import lax
from jax.experimental import pallas as pl
from jax.experimental.pallas import tpu as pltpu
from jax.experimental.pallas import tpu_sc as plsc

F32 = jnp.float32
I32 = jnp.int32

N0 = 10000
N0P = 10240
E1 = 160000
E2 = 65536
N1 = 4096
N2 = 1024


def _proj_body(x_ref, wl_ref, wr_ref, out_l_ref, out_r_ref):
    x = x_ref[...]
    out_l_ref[...] = jnp.dot(x, wl_ref[...], preferred_element_type=F32)
    out_r_ref[...] = jnp.dot(x, wr_ref[...], preferred_element_type=F32)


def _proj(x, wl, wr):
    return pl.pallas_call(
        _proj_body,
        grid=(2,),
        in_specs=[
            pl.BlockSpec((5000, 256), lambda i: (i, 0)),
            pl.BlockSpec((256, 16), lambda i: (0, 0)),
            pl.BlockSpec((256, 16), lambda i: (0, 0)),
        ],
        out_specs=[
            pl.BlockSpec((5000, 16), lambda i: (i, 0)),
            pl.BlockSpec((5000, 16), lambda i: (i, 0)),
        ],
        out_shape=[
            jax.ShapeDtypeStruct((N0, 16), F32),
            jax.ShapeDtypeStruct((N0, 16), F32),
        ],
    )(x, wl, wr)


_MESH = plsc.VectorSubcoreMesh(core_axis_name="c", subcore_axis_name="s")
_SC_PARAMS = pltpu.CompilerParams(use_tc_tiling_on_sc=False,
                                  needs_layout_passes=False)


@functools.partial(
    pl.kernel,
    mesh=_MESH,
    out_type=[
        jax.ShapeDtypeStruct((N1, 16), F32),
        jax.ShapeDtypeStruct((2, N0P, 16), F32),
        jax.ShapeDtypeStruct((2, N1, 16), F32),
        jax.ShapeDtypeStruct((2, N1), F32),
    ],
    scratch_types=[
        pltpu.VMEM((256, 16), F32),
        pltpu.VMEM((256,), F32),
        pltpu.VMEM((1024,), F32),
        pltpu.VMEM((1024,), I32),
        pltpu.VMEM((1024,), I32),
        pltpu.VMEM((1000, 16), F32),
        pltpu.VMEM((1000, 16), F32),
        pltpu.VMEM((1024, 16), F32),
        pltpu.VMEM((5000,), I32),
        pltpu.VMEM((5000,), I32),
        pltpu.VMEM_SHARED((N1, 16), F32),
        pltpu.VMEM_SHARED((N1,), F32),
        pltpu.SemaphoreType.DMA,
        pltpu.SemaphoreType.DMA,
        pltpu.SemaphoreType.DMA,
        pltpu.SemaphoreType.DMA,
        pltpu.SemaphoreType.DMA,
    ],
    compiler_params=_SC_PARAMS,
)
def _sc1(pl_hbm, pr_hbm, nid_hbm, src_hbm, dst_hbm,
         root_out, g_out, acc_out, cnt_out,
         zbuf, czbuf, ones_v, nid_v, ridx_v, growA, growB, rrow_v,
         src_v, dst_v, acc_sh, cnt_sh,
         sem_g0, sem_g1, sem_a0, sem_a1, sem_c):
    c = lax.axis_index("c")
    s = lax.axis_index("s")
    w = s * 2 + c

    def fillz(i, _):
        zbuf[i] = jnp.zeros((16,), F32)
        return 0
    lax.fori_loop(0, 256, fillz, 0)

    def fillcz(i, _):
        czbuf[pl.ds(i * 16, 16)] = jnp.zeros((16,), F32)
        return 0
    lax.fori_loop(0, 16, fillcz, 0)

    def fill1(i, _):
        ones_v[pl.ds(i * 16, 16)] = jnp.full((16,), 1.0, F32)
        return 0
    lax.fori_loop(0, 64, fill1, 0)

    zd0 = pltpu.async_copy(zbuf, acc_sh.at[pl.ds(s * 256, 256)], sem_a0)
    zd1 = pltpu.async_copy(czbuf, cnt_sh.at[pl.ds(s * 256, 256)], sem_a1)

    pd0 = pltpu.async_copy(src_hbm.at[pl.ds(w * 5000, 5000)], src_v, sem_g0)
    pd1 = pltpu.async_copy(dst_hbm.at[pl.ds(w * 5000, 5000)], dst_v, sem_g1)

    @pl.when(s < 9)
    def _():
        pltpu.sync_copy(nid_hbm.at[pl.ds(s * 1024, 1024)], nid_v)
        pltpu.sync_copy(pl_hbm.at[nid_v], rrow_v)
        pltpu.sync_copy(rrow_v, g_out.at[c, pl.ds(s * 1024, 1024)])

    @pl.when(s == 9)
    def _():
        pltpu.sync_copy(nid_hbm.at[pl.ds(9216, 784)], nid_v.at[pl.ds(0, 784)])
        pltpu.sync_copy(pl_hbm.at[nid_v.at[pl.ds(0, 784)]],
                        rrow_v.at[pl.ds(0, 784)])
        pltpu.sync_copy(rrow_v.at[pl.ds(0, 784)],
                        g_out.at[c, pl.ds(9216, 784)])

    @pl.when((s >= 10) & (s < 12))
    def _():
        q = (s - 10) * 2 + c
        pltpu.sync_copy(nid_hbm.at[pl.ds(q * 1024, 1024)], ridx_v)
        pltpu.sync_copy(pr_hbm.at[ridx_v], rrow_v)
        pltpu.sync_copy(rrow_v, root_out.at[pl.ds(q * 1024, 1024)])

    zd0.wait()
    zd1.wait()
    pd0.wait()
    pd1.wait()
    plsc.subcore_barrier()

    gtab = g_out.at[c]
    bufs = (growA, growB)
    gsems = (sem_g0, sem_g1)
    asems = (sem_a0, sem_a1)
    gd = [None] * 5
    ad = [None] * 5
    cd = [None] * 5
    gd[0] = pltpu.async_copy(gtab.at[src_v.at[pl.ds(0, 1000)]], bufs[0],
                             gsems[0])
    for j in range(5):
        if j + 1 < 5:
            if j - 1 >= 0:
                ad[j - 1].wait()
            gd[j + 1] = pltpu.async_copy(
                gtab.at[src_v.at[pl.ds((j + 1) * 1000, 1000)]],
                bufs[(j + 1) % 2], gsems[(j + 1) % 2])
        gd[j].wait()
        ad[j] = pltpu.async_copy(
            bufs[j % 2], acc_sh.at[dst_v.at[pl.ds(j * 1000, 1000)]],
            asems[j % 2], add=True)
        cd[j] = pltpu.async_copy(
            ones_v.at[pl.ds(0, 1000)],
            cnt_sh.at[dst_v.at[pl.ds(j * 1000, 1000)]], sem_c, add=True)
    ad[3].wait()
    ad[4].wait()
    for j in range(5):
        cd[j].wait()

    plsc.subcore_barrier()

    pltpu.sync_copy(acc_sh.at[pl.ds(s * 256, 256)], growA.at[pl.ds(0, 256)])
    pltpu.sync_copy(growA.at[pl.ds(0, 256)], acc_out.at[c, pl.ds(s * 256, 256)])
    pltpu.sync_copy(cnt_sh.at[pl.ds(s * 256, 256)], czbuf)
    pltpu.sync_copy(czbuf, cnt_out.at[c, pl.ds(s * 256, 256)])


@functools.partial(
    pl.kernel,
    mesh=_MESH,
    out_type=[
        jax.ShapeDtypeStruct((5, N2, 16), F32),
        jax.ShapeDtypeStruct((2, N1, 16), F32),
    ],
    scratch_types=[
        pltpu.VMEM((64, 16), F32),
        pltpu.VMEM((64,), F32),
        pltpu.VMEM((256, 16), F32),
        pltpu.VMEM((256, 16), F32),
        pltpu.VMEM((256,), F32),
        pltpu.VMEM((256,), F32),
        pltpu.VMEM((256, 16), F32),
        pltpu.VMEM((256, 16), F32),
        pltpu.VMEM((16,), F32),
        pltpu.VMEM((1024, 16), F32),
        pltpu.VMEM((1024, 16), F32),
        pltpu.VMEM((2048,), I32),
        pltpu.VMEM((2048,), I32),
        pltpu.VMEM((1024,), F32),
        pltpu.VMEM_SHARED((N2, 16), F32),
        pltpu.VMEM_SHARED((N2,), F32),
        pltpu.SemaphoreType.DMA,
        pltpu.SemaphoreType.DMA,
        pltpu.SemaphoreType.DMA,
        pltpu.SemaphoreType.DMA,
        pltpu.SemaphoreType.DMA,
    ],
    compiler_params=_SC_PARAMS,
)
def _sc2(acc1_hbm, cnt1_hbm, root_hbm, b_hbm, src_hbm, dst_hbm,
         l2_out, h1c_out,
         zbuf, czbuf, a0_v, a1_v, c0_v, c1_v, root_v, hbuf, b_v,
         growA, growB, src_v, dst_v, ones_v, acc_sh, cnt_sh,
         sem_g0, sem_g1, sem_a0, sem_a1, sem_c):
    c = lax.axis_index("c")
    s = lax.axis_index("s")
    w = s * 2 + c

    def fill(i, _):
        zbuf[i] = jnp.zeros((16,), F32)
        return 0
    lax.fori_loop(0, 64, fill, 0)

    def fillcz(i, _):
        czbuf[pl.ds(i * 16, 16)] = jnp.zeros((16,), F32)
        return 0
    lax.fori_loop(0, 4, fillcz, 0)

    def fill1(i, _):
        ones_v[pl.ds(i * 16, 16)] = jnp.full((16,), 1.0, F32)
        return 0
    lax.fori_loop(0, 64, fill1, 0)

    zd0 = pltpu.async_copy(zbuf, acc_sh.at[pl.ds(s * 64, 64)], sem_a0)
    zd1 = pltpu.async_copy(czbuf, cnt_sh.at[pl.ds(s * 64, 64)], sem_a1)

    r0 = s * 256
    d0 = pltpu.async_copy(acc1_hbm.at[0, pl.ds(r0, 256)], a0_v, sem_c)
    d1 = pltpu.async_copy(acc1_hbm.at[1, pl.ds(r0, 256)], a1_v, sem_c)
    d2 = pltpu.async_copy(cnt1_hbm.at[0, pl.ds(r0, 256)], c0_v, sem_c)
    d3 = pltpu.async_copy(cnt1_hbm.at[1, pl.ds(r0, 256)], c1_v, sem_c)
    d4 = pltpu.async_copy(root_hbm.at[pl.ds(r0, 256)], root_v, sem_c)
    d5 = pltpu.async_copy(b_hbm, b_v, sem_c)
    for d in (d0, d1, d2, d3, d4, d5):
        d.wait()

    def cadd(i, _):
        o = pl.ds(i * 16, 16)
        c0_v[o] = c0_v[o] + c1_v[o]
        return 0
    lax.fori_loop(0, 16, cadd, 0)

    def hrow(i, _):
        n = jnp.maximum(plsc.load_gather(c0_v, [jnp.full((16,), i, I32)]), 1.0)
        h = jnp.maximum((a0_v[i] + a1_v[i]) / n + root_v[i] + b_v[...], 0.0)
        hbuf[i] = h
        return 0
    lax.fori_loop(0, 256, hrow, 0)

    hw = pltpu.async_copy(hbuf, h1c_out.at[c, pl.ds(r0, 256)], sem_c)

    @pl.when((s < 4) & (c == 0))
    def _():
        pltpu.sync_copy(hbuf, l2_out.at[4, pl.ds(r0, 256)])

    zd0.wait()
    zd1.wait()
    hw.wait()
    plsc.subcore_barrier()

    pltpu.sync_copy(src_hbm.at[pl.ds(w * 2048, 2048)], src_v)
    pltpu.sync_copy(dst_hbm.at[pl.ds(w * 2048, 2048)], dst_v)

    htab = h1c_out.at[c]
    g0 = pltpu.async_copy(htab.at[src_v.at[pl.ds(0, 1024)]], growA, sem_g0)
    g1 = pltpu.async_copy(htab.at[src_v.at[pl.ds(1024, 1024)]], growB, sem_g1)
    g0.wait()
    a0 = pltpu.async_copy(growA, acc_sh.at[dst_v.at[pl.ds(0, 1024)]],
                          sem_a0, add=True)
    k0 = pltpu.async_copy(ones_v, cnt_sh.at[dst_v.at[pl.ds(0, 1024)]],
                          sem_c, add=True)
    g1.wait()
    a1 = pltpu.async_copy(growB, acc_sh.at[dst_v.at[pl.ds(1024, 1024)]],
                          sem_a1, add=True)
    k1 = pltpu.async_copy(ones_v, cnt_sh.at[dst_v.at[pl.ds(1024, 1024)]],
                          sem_c, add=True)
    for d in (a0, k0, a1, k1):
        d.wait()

    plsc.subcore_barrier()

    r = s * 64
    pltpu.sync_copy(acc_sh.at[pl.ds(r, 64)], growA.at[pl.ds(0, 64)])
    pltpu.sync_copy(growA.at[pl.ds(0, 64)], l2_out.at[c, pl.ds(r, 64)])

    pltpu.sync_copy(cnt_sh.at[pl.ds(r, 64)], czbuf)

    def cbc(i, _):
        growB[i] = plsc.load_gather(czbuf, [jnp.full((16,), i, I32)])
        return 0
    lax.fori_loop(0, 64, cbc, 0)
    pltpu.sync_copy(growB.at[pl.ds(0, 64)], l2_out.at[2 + c, pl.ds(r, 64)])


def _comb2_body(l2_ref, wl_ref, wr_ref, b_ref, out_ref):
    sums = l2_ref[0] + l2_ref[1]
    cnt = jnp.maximum(l2_ref[2] + l2_ref[3], 1.0)
    mean = sums / cnt
    z = (jnp.dot(mean, wl_ref[...], preferred_element_type=F32)
         + jnp.dot(l2_ref[4], wr_ref[...], preferred_element_type=F32)
         + b_ref[...][None, :])
    m = jnp.max(z, axis=1, keepdims=True)
    lse = m + jnp.log(jnp.sum(jnp.exp(z - m), axis=1, keepdims=True))
    out_ref[...] = z - lse


def _comb2(l2, wl2, wr2, b2):
    return pl.pallas_call(
        _comb2_body,
        out_shape=jax.ShapeDtypeStruct((N2, 41), F32),
    )(l2, wl2, wr2, b2)


def kernel(x, n_id, src1, dst1, src2, dst2, W_l1, W_r1, b1, W_l2, W_r2, b2):
    p_l, p_r = _proj(x, W_l1, W_r1)

    root1, _g, acc1, cnt1 = _sc1(p_l, p_r, n_id.astype(I32),
                                 src1.astype(I32), dst1.astype(I32))

    l2, _h1c = _sc2(acc1, cnt1, root1, b1,
                    src2.astype(I32), dst2.astype(I32))

    return _comb2(l2, W_l2, W_r2, b2)
